# baseline (speedup 1.0000x reference)
import functools

import jax
import jax.numpy as jnp
from jax import lax
from jax.experimental import pallas as pl
from jax.experimental.pallas import tpu as pltpu
from jax.experimental.pallas import tpu_sc as plsc

_NC, _NS = 2, 16
_NW = _NC * _NS
_CH = 16
_LANES = 16
_NXB = 4
_LOOKAHEAD = 3


def kernel(x, pe):
    B, S, D = x.shape
    pes = pe[:S]
    rows_per_w = S // _NW
    n_chunks = rows_per_w // _CH
    n_steps = n_chunks * B

    @functools.partial(
        pl.kernel,
        out_type=jax.ShapeDtypeStruct((B, S, D), jnp.float32),
        mesh=plsc.VectorSubcoreMesh(core_axis_name="c", subcore_axis_name="s"),
        scratch_types=[
            pltpu.VMEM((_CH, D), jnp.float32),
            pltpu.VMEM((_CH, D), jnp.float32),
            pltpu.VMEM((_CH, D), jnp.float32),
            pltpu.VMEM((_CH, D), jnp.float32),
            pltpu.VMEM((_CH, D), jnp.float32),
            pltpu.VMEM((_CH, D), jnp.float32),
            pltpu.SemaphoreType.DMA,
            pltpu.SemaphoreType.DMA,
            pltpu.SemaphoreType.DMA,
            pltpu.SemaphoreType.DMA,
            pltpu.SemaphoreType.DMA,
            pltpu.SemaphoreType.DMA,
            pltpu.SemaphoreType.DMA,
            pltpu.SemaphoreType.DMA,
            pltpu.SemaphoreType.DMA,
            pltpu.SemaphoreType.DMA,
        ],
    )
    def _sc_pe_add(x_hbm, pe_hbm, out_hbm,
                   pbuf0, pbuf1, xbuf0, xbuf1, xbuf2, xbuf3,
                   psem0, psem1, xsem0, xsem1, xsem2, xsem3,
                   osem0, osem1, osem2, osem3):
        pbuf = [pbuf0, pbuf1]
        xbuf = [xbuf0, xbuf1, xbuf2, xbuf3]
        psem = [psem0, psem1]
        xsem = [xsem0, xsem1, xsem2, xsem3]
        osem = [osem0, osem1, osem2, osem3]

        wid = lax.axis_index("s") * _NC + lax.axis_index("c")
        s_base = wid * rows_per_w

        def s0_of(c):
            return s_base + c * _CH

        def add_pe(xb, pb):
            @plsc.parallel_loop(0, _CH * D, _LANES, unroll=8)
            def _(i):
                r = i >> (D.bit_length() - 1)
                col = pl.multiple_of(i & (D - 1), _LANES)
                plsc.addupdate(xb.at[r, pl.ds(col, _LANES)],
                               pb[r, pl.ds(col, _LANES)])

        pe_h = [None] * n_chunks
        x_h = [None] * n_steps
        o_h = [None] * n_steps

        pe_h[0] = pltpu.async_copy(
            pe_hbm.at[pl.ds(s0_of(0), _CH)], pbuf[0], psem[0])
        for t in range(min(_LOOKAHEAD, n_steps)):
            c, b = divmod(t, B)
            x_h[t] = pltpu.async_copy(
                x_hbm.at[b, pl.ds(s0_of(c), _CH)],
                xbuf[t % _NXB], xsem[t % _NXB])

        for t in range(n_steps):
            c, b = divmod(t, B)
            if b == 0:
                if c + 1 < n_chunks:
                    pe_h[c + 1] = pltpu.async_copy(
                        pe_hbm.at[pl.ds(s0_of(c + 1), _CH)],
                        pbuf[(c + 1) % 2], psem[(c + 1) % 2])
                pe_h[c].wait()
            tt = t + _LOOKAHEAD
            if tt < n_steps:
                cc, bb = divmod(tt, B)
                if tt - _NXB >= 0:
                    o_h[tt - _NXB].wait()
                x_h[tt] = pltpu.async_copy(
                    x_hbm.at[bb, pl.ds(s0_of(cc), _CH)],
                    xbuf[tt % _NXB], xsem[tt % _NXB])
            x_h[t].wait()
            add_pe(xbuf[t % _NXB], pbuf[c % 2])
            o_h[t] = pltpu.async_copy(
                xbuf[t % _NXB], out_hbm.at[b, pl.ds(s0_of(c), _CH)],
                osem[t % _NXB])

        for t in range(max(0, n_steps - _NXB), n_steps):
            o_h[t].wait()

    return _sc_pe_add(x, pes)


# speedup vs baseline: 1.1110x; 1.1110x over previous
import functools

import jax
import jax.numpy as jnp
from jax import lax
from jax.experimental import pallas as pl
from jax.experimental.pallas import tpu as pltpu
from jax.experimental.pallas import tpu_sc as plsc

_NC, _NS = 2, 16
_NW = _NC * _NS
_CH = 16
_LANES = 16
_NXB = 4
_LOOKAHEAD = 2


def kernel(x, pe):
    B, S, D = x.shape
    pes = pe[:S]
    rows_per_w = S // _NW
    n_chunks = rows_per_w // _CH
    n_steps = n_chunks * B

    @functools.partial(
        pl.kernel,
        out_type=jax.ShapeDtypeStruct((B, S, D), jnp.float32),
        mesh=plsc.VectorSubcoreMesh(core_axis_name="c", subcore_axis_name="s"),
        scratch_types=[
            pltpu.VMEM((_CH, D), jnp.float32),
            pltpu.VMEM((_CH, D), jnp.float32),
            pltpu.VMEM((_CH, D), jnp.float32),
            pltpu.VMEM((_CH, D), jnp.float32),
            pltpu.VMEM((_CH, D), jnp.float32),
            pltpu.VMEM((_CH, D), jnp.float32),
            pltpu.SemaphoreType.DMA,
            pltpu.SemaphoreType.DMA,
            pltpu.SemaphoreType.DMA,
            pltpu.SemaphoreType.DMA,
            pltpu.SemaphoreType.DMA,
            pltpu.SemaphoreType.DMA,
            pltpu.SemaphoreType.DMA,
            pltpu.SemaphoreType.DMA,
            pltpu.SemaphoreType.DMA,
            pltpu.SemaphoreType.DMA,
        ],
    )
    def _sc_pe_add(x_hbm, pe_hbm, out_hbm,
                   pbuf0, pbuf1, xbuf0, xbuf1, xbuf2, xbuf3,
                   psem0, psem1, xsem0, xsem1, xsem2, xsem3,
                   osem0, osem1, osem2, osem3):
        pbuf = [pbuf0, pbuf1]
        xbuf = [xbuf0, xbuf1, xbuf2, xbuf3]
        psem = [psem0, psem1]
        xsem = [xsem0, xsem1, xsem2, xsem3]
        osem = [osem0, osem1, osem2, osem3]

        wid = lax.axis_index("s") * _NC + lax.axis_index("c")
        s_base = wid * rows_per_w

        def s0_of(c):
            return s_base + c * _CH

        def add_pe(xb, pb):
            @plsc.parallel_loop(0, _CH * D, _LANES, unroll=8)
            def _(i):
                r = i >> (D.bit_length() - 1)
                col = pl.multiple_of(i & (D - 1), _LANES)
                plsc.addupdate(xb.at[r, pl.ds(col, _LANES)],
                               pb[r, pl.ds(col, _LANES)])

        pe_h = [None] * n_chunks
        x_h = [None] * n_steps
        o_h = [None] * n_steps

        pe_h[0] = pltpu.async_copy(
            pe_hbm.at[pl.ds(s0_of(0), _CH)], pbuf[0], psem[0])
        for t in range(min(_LOOKAHEAD, n_steps)):
            c, b = divmod(t, B)
            x_h[t] = pltpu.async_copy(
                x_hbm.at[b, pl.ds(s0_of(c), _CH)],
                xbuf[t % _NXB], xsem[t % _NXB])

        for t in range(n_steps):
            c, b = divmod(t, B)
            if b == 0:
                if c + 1 < n_chunks:
                    pe_h[c + 1] = pltpu.async_copy(
                        pe_hbm.at[pl.ds(s0_of(c + 1), _CH)],
                        pbuf[(c + 1) % 2], psem[(c + 1) % 2])
                pe_h[c].wait()
            tt = t + _LOOKAHEAD
            if tt < n_steps:
                cc, bb = divmod(tt, B)
                if tt - _NXB >= 0:
                    o_h[tt - _NXB].wait()
                x_h[tt] = pltpu.async_copy(
                    x_hbm.at[bb, pl.ds(s0_of(cc), _CH)],
                    xbuf[tt % _NXB], xsem[tt % _NXB])
            x_h[t].wait()
            add_pe(xbuf[t % _NXB], pbuf[c % 2])
            o_h[t] = pltpu.async_copy(
                xbuf[t % _NXB], out_hbm.at[b, pl.ds(s0_of(c), _CH)],
                osem[t % _NXB])

        for t in range(max(0, n_steps - _NXB), n_steps):
            o_h[t].wait()

    return _sc_pe_add(x, pes)
